# baseline (speedup 1.0000x reference)
import functools

import jax
import jax.numpy as jnp
from jax import lax
from jax.experimental import pallas as pl
from jax.experimental.pallas import tpu as pltpu
from jax.experimental.pallas import tpu_sc as plsc

NN = 100000
EE = 3200000
HID = 32
GG = 64

NC, NS = 2, 16
NW = NC * NS
NPAD = 100352
STRIPE = NPAD // NS
ROWS2D = NPAD // 128
RPW = 784
KT = 56
NT = RPW // KT
EPW = RPW * 128
EPAD = NW * EPW

RBLK = 2048
NSTEPS = NPAD // RBLK

_f32 = jnp.float32


def _zero_acc(sid, ztile, acc):
    @pl.loop(0, STRIPE, step=16)
    def _(i):
        ztile[pl.ds(i, 16)] = jnp.zeros((16,), _f32)

    pltpu.sync_copy(ztile, acc.at[pl.ds(sid * STRIPE, STRIPE)])


def _writeback(cid, sid, acc, out_hbm):
    pltpu.sync_copy(acc.at[pl.ds(sid * STRIPE, STRIPE)],
                    out_hbm.at[cid, pl.ds(sid * STRIPE, STRIPE)])


def _fire_scatters(val, idx, acc, sem):
    @pl.loop(0, KT)
    def _(j):
        pltpu.async_copy(val.at[j], acc.at[idx.at[j]], sem, add=True)


def _drain_scatters(val, idx, acc, sem):
    @pl.loop(0, KT)
    def _(j):
        pltpu.make_async_copy(val.at[j], acc.at[idx.at[j]], sem).wait()


def _fire_gathers(tab_hbm, idx, val, sem):
    @pl.loop(0, KT)
    def _(j):
        pltpu.async_copy(tab_hbm.at[idx.at[j]], val.at[j], sem)


def _drain_gathers(tab_hbm, idx, val, sem):
    @pl.loop(0, KT)
    def _(j):
        pltpu.make_async_copy(tab_hbm.at[idx.at[j]], val.at[j], sem).wait()


def _sc_hist(dst_hbm, out_hbm, idxa, idxb, ones_t, ztile, acc, sema, semb):
    cid = lax.axis_index("c")
    sid = lax.axis_index("s")
    wid = cid * NS + sid

    @pl.loop(0, 128, step=16)
    def _(i):
        ones_t[pl.ds(i, 16)] = jnp.ones((16,), _f32)

    _zero_acc(sid, ztile, acc)
    plsc.subcore_barrier()

    @pl.loop(0, NT, step=2)
    def _(t):
        pltpu.sync_copy(dst_hbm.at[wid, pl.ds(t * KT, KT)], idxa)

        @pl.loop(0, KT)
        def _(j):
            pltpu.async_copy(ones_t, acc.at[idxa.at[j]], sema, add=True)

        pltpu.sync_copy(dst_hbm.at[wid, pl.ds((t + 1) * KT, KT)], idxb)

        @pl.loop(0, KT)
        def _(j):
            pltpu.async_copy(ones_t, acc.at[idxb.at[j]], semb, add=True)

        @pl.loop(0, KT)
        def _(j):
            pltpu.make_async_copy(ones_t, acc.at[idxa.at[j]], sema).wait()

        @pl.loop(0, KT)
        def _(j):
            pltpu.make_async_copy(ones_t, acc.at[idxb.at[j]], semb).wait()

    plsc.subcore_barrier()
    _writeback(cid, sid, acc, out_hbm)


def _sc_agg1(src_hbm, dst_hbm, y_hbm, out_hbm,
             sidxa, sidxb, didxa, didxb, vala, valb, ztile, acc, ysh,
             gsema, gsemb, ssema, ssemb):
    cid = lax.axis_index("c")
    sid = lax.axis_index("s")
    wid = cid * NS + sid

    _zero_acc(sid, ztile, acc)
    pltpu.sync_copy(y_hbm.at[pl.ds(sid * STRIPE, STRIPE)], ztile)
    pltpu.sync_copy(ztile, ysh.at[pl.ds(sid * STRIPE, STRIPE)])
    plsc.subcore_barrier()

    @pl.loop(0, NT, step=2)
    def _(t):
        pltpu.sync_copy(src_hbm.at[wid, pl.ds(t * KT, KT)], sidxa)
        pltpu.sync_copy(dst_hbm.at[wid, pl.ds(t * KT, KT)], didxa)
        _fire_gathers(ysh, sidxa, vala, gsema)
        pltpu.sync_copy(src_hbm.at[wid, pl.ds((t + 1) * KT, KT)], sidxb)
        pltpu.sync_copy(dst_hbm.at[wid, pl.ds((t + 1) * KT, KT)], didxb)
        _fire_gathers(ysh, sidxb, valb, gsemb)
        _drain_gathers(ysh, sidxa, vala, gsema)
        _fire_scatters(vala, didxa, acc, ssema)
        _drain_gathers(ysh, sidxb, valb, gsemb)
        _fire_scatters(valb, didxb, acc, ssemb)
        _drain_scatters(vala, didxa, acc, ssema)
        _drain_scatters(valb, didxb, acc, ssemb)

    plsc.subcore_barrier()
    _writeback(cid, sid, acc, out_hbm)


def _sc_agg2(src_hbm, dst_hbm, g_hbm, outu_hbm, outv_hbm,
             sidxa, sidxb, didxa, didxb, valga, valgb, valza, valzb,
             ztile, accu, accv, gsh, gsema, gsemb, ssema, ssemb):
    cid = lax.axis_index("c")
    sid = lax.axis_index("s")
    wid = cid * NS + sid

    _zero_acc(sid, ztile, accu)
    _zero_acc(sid, ztile, accv)
    pltpu.sync_copy(g_hbm.at[pl.ds(sid * STRIPE, STRIPE)], ztile)
    pltpu.sync_copy(ztile, gsh.at[pl.ds(sid * STRIPE, STRIPE)])
    plsc.subcore_barrier()

    def relu_pair(valg, valz):
        @pl.loop(0, KT)
        def _(j):
            @pl.loop(0, 128, step=16)
            def _(i):
                vg = valg.at[j][pl.ds(i, 16)]
                valz.at[j][pl.ds(i, 16)] = jnp.maximum(-vg, 0.0)
                valg.at[j][pl.ds(i, 16)] = jnp.maximum(vg, 0.0)

    @pl.loop(0, NT, step=2)
    def _(t):
        pltpu.sync_copy(src_hbm.at[wid, pl.ds(t * KT, KT)], sidxa)
        pltpu.sync_copy(dst_hbm.at[wid, pl.ds(t * KT, KT)], didxa)
        _fire_gathers(gsh, sidxa, valga, gsema)
        pltpu.sync_copy(src_hbm.at[wid, pl.ds((t + 1) * KT, KT)], sidxb)
        pltpu.sync_copy(dst_hbm.at[wid, pl.ds((t + 1) * KT, KT)], didxb)
        _fire_gathers(gsh, sidxb, valgb, gsemb)
        _drain_gathers(gsh, sidxa, valga, gsema)
        relu_pair(valga, valza)
        _fire_scatters(valga, didxa, accu, ssema)
        _fire_scatters(valza, didxa, accv, ssema)
        _drain_gathers(gsh, sidxb, valgb, gsemb)
        relu_pair(valgb, valzb)
        _fire_scatters(valgb, didxb, accu, ssemb)
        _fire_scatters(valzb, didxb, accv, ssemb)
        _drain_scatters(valga, didxa, accu, ssema)
        _drain_scatters(valza, didxa, accv, ssema)
        _drain_scatters(valgb, didxb, accu, ssemb)
        _drain_scatters(valzb, didxb, accv, ssemb)

    plsc.subcore_barrier()
    _writeback(cid, sid, accu, outu_hbm)
    _writeback(cid, sid, accv, outv_hbm)


@functools.lru_cache(maxsize=1)
def _sc_kernels():
    mesh = plsc.VectorSubcoreMesh(core_axis_name="c", subcore_axis_name="s",
                                  num_cores=NC, num_subcores=NS)
    one_out = jax.ShapeDtypeStruct((NC, NPAD), _f32)
    idx_t = pltpu.VMEM((KT, 128), jnp.int32)
    val_t = pltpu.VMEM((KT, 128), _f32)
    dma = pltpu.SemaphoreType.DMA
    hist = pl.kernel(
        _sc_hist, out_type=one_out, mesh=mesh,
        scratch_types=[
            idx_t, idx_t,
            pltpu.VMEM((128,), _f32),
            pltpu.VMEM((STRIPE,), _f32),
            pltpu.VMEM_SHARED((NPAD,), _f32),
            dma, dma,
        ])
    agg1 = pl.kernel(
        _sc_agg1, out_type=one_out, mesh=mesh,
        scratch_types=[
            idx_t, idx_t, idx_t, idx_t,
            val_t, val_t,
            pltpu.VMEM((STRIPE,), _f32),
            pltpu.VMEM_SHARED((NPAD,), _f32),
            pltpu.VMEM_SHARED((NPAD,), _f32),
            dma, dma, dma, dma,
        ])
    agg2 = pl.kernel(
        _sc_agg2, out_type=[one_out, one_out], mesh=mesh,
        scratch_types=[
            idx_t, idx_t, idx_t, idx_t,
            val_t, val_t, val_t, val_t,
            pltpu.VMEM((STRIPE,), _f32),
            pltpu.VMEM_SHARED((NPAD,), _f32),
            pltpu.VMEM_SHARED((NPAD,), _f32),
            pltpu.VMEM_SHARED((NPAD,), _f32),
            dma, dma, dma, dma,
        ])
    return hist, agg1, agg2


def _tc_prep_body(degp, x2d, dinv_o, invd_o, y_o):
    deg = degp[0] + degp[1] + 1.0
    dinv = lax.rsqrt(deg)
    invd = 1.0 / deg
    dinv_o[...] = dinv
    invd_o[...] = invd
    y_o[...] = dinv * x2d[...]


def _tc_uvwz_body(tp, dinv, invd, x2d, u_o, v_o, g_o):
    t = dinv[...] * (tp[0] + tp[1]) + x2d[...] * invd[...]
    u_o[...] = jnp.maximum(t, 0.0)
    v_o[...] = jnp.maximum(-t, 0.0)
    g_o[...] = dinv[...] * t


def _tc_pool_body(up, vp, dinv, invd, u, v, bat, a, c, b2,
                  wl1, bl1, wl2, bl2, out, sums, cnt):
    i = pl.program_id(0)

    @pl.when(i == 0)
    def _():
        sums[...] = jnp.zeros_like(sums)
        cnt[...] = jnp.zeros_like(cnt)

    uu = dinv[...] * (up[0] + up[1]) + u[...] * invd[...]
    vv = dinv[...] * (vp[0] + vp[1]) + v[...] * invd[...]
    h2 = jnp.maximum(uu * a[...] + vv * c[...] + b2[...], 0.0)
    seg = lax.broadcasted_iota(jnp.int32, (RBLK, GG), 1)
    onehot = (bat[...] == seg).astype(_f32)
    sums[...] += lax.dot_general(onehot, h2, (((0,), (0,)), ((), ())),
                                 preferred_element_type=_f32)
    cnt[...] += lax.dot_general(onehot, jnp.ones((RBLK, 1), _f32),
                                (((0,), (0,)), ((), ())),
                                preferred_element_type=_f32)

    @pl.when(i == NSTEPS - 1)
    def _():
        pooled = sums[...] / jnp.maximum(cnt[...], 1.0)
        hh = jnp.maximum(
            jnp.dot(pooled, wl1[...], preferred_element_type=_f32)
            + bl1[...], 0.0)
        oo = jnp.dot(hh, wl2[...], preferred_element_type=_f32) + bl2[...]
        out[...] = jax.nn.sigmoid(oo)


def kernel(x, edge_index, batch, W1, b1, W2, b2, Wl1, bl1, Wl2, bl2):
    f32 = _f32
    src = edge_index[0].astype(jnp.int32)
    dst = edge_index[1].astype(jnp.int32)
    npad_e = EPAD - EE
    src3 = jnp.concatenate([src, jnp.zeros((npad_e,), jnp.int32)]) \
        .reshape(NW, RPW, 128)
    dst3 = jnp.concatenate([dst, jnp.full((npad_e,), NN, jnp.int32)]) \
        .reshape(NW, RPW, 128)

    xp = jnp.concatenate([x[:, 0].astype(f32),
                          jnp.zeros((NPAD - NN,), f32)])
    x2d = xp.reshape(ROWS2D, 128)

    w1row = W1[0].astype(f32)
    p = jnp.maximum(w1row, 0.0)
    q = jnp.maximum(-w1row, 0.0)
    a = (p @ W2.astype(f32)).reshape(1, HID)
    c = (q @ W2.astype(f32)).reshape(1, HID)
    b2r = b2.astype(f32).reshape(1, HID)

    sc_hist, sc_agg1, sc_agg2 = _sc_kernels()

    degp = sc_hist(dst3)

    dinv2d, invd2d, y2d = pl.pallas_call(
        _tc_prep_body,
        out_shape=[jax.ShapeDtypeStruct((ROWS2D, 128), f32)] * 3,
    )(degp.reshape(NC, ROWS2D, 128), x2d)

    tp = sc_agg1(src3, dst3, y2d.reshape(NPAD))

    u2d, v2d, g2d = pl.pallas_call(
        _tc_uvwz_body,
        out_shape=[jax.ShapeDtypeStruct((ROWS2D, 128), f32)] * 3,
    )(tp.reshape(NC, ROWS2D, 128), dinv2d, invd2d, x2d)

    up, vp = sc_agg2(src3, dst3, g2d.reshape(NPAD))

    batp = jnp.concatenate([batch.astype(jnp.int32),
                            jnp.full((NPAD - NN,), GG, jnp.int32)]) \
        .reshape(NPAD, 1)

    out = pl.pallas_call(
        _tc_pool_body,
        grid=(NSTEPS,),
        in_specs=[
            pl.BlockSpec((NC, RBLK, 1), lambda i: (0, i, 0)),
            pl.BlockSpec((NC, RBLK, 1), lambda i: (0, i, 0)),
            pl.BlockSpec((RBLK, 1), lambda i: (i, 0)),
            pl.BlockSpec((RBLK, 1), lambda i: (i, 0)),
            pl.BlockSpec((RBLK, 1), lambda i: (i, 0)),
            pl.BlockSpec((RBLK, 1), lambda i: (i, 0)),
            pl.BlockSpec((RBLK, 1), lambda i: (i, 0)),
            pl.BlockSpec((1, HID), lambda i: (0, 0)),
            pl.BlockSpec((1, HID), lambda i: (0, 0)),
            pl.BlockSpec((1, HID), lambda i: (0, 0)),
            pl.BlockSpec((HID, 16), lambda i: (0, 0)),
            pl.BlockSpec((1, 16), lambda i: (0, 0)),
            pl.BlockSpec((16, 1), lambda i: (0, 0)),
            pl.BlockSpec((1, 1), lambda i: (0, 0)),
        ],
        out_specs=pl.BlockSpec((GG, 1), lambda i: (0, 0)),
        out_shape=jax.ShapeDtypeStruct((GG, 1), f32),
        scratch_shapes=[pltpu.VMEM((GG, HID), f32),
                        pltpu.VMEM((GG, 1), f32)],
    )(up.reshape(NC, NPAD, 1), vp.reshape(NC, NPAD, 1),
      dinv2d.reshape(NPAD, 1), invd2d.reshape(NPAD, 1),
      u2d.reshape(NPAD, 1), v2d.reshape(NPAD, 1), batp,
      a, c, b2r,
      Wl1.astype(f32), bl1.astype(f32).reshape(1, 16),
      Wl2.astype(f32), bl2.astype(f32).reshape(1, 1))

    return out.reshape(GG)


# speedup vs baseline: 1.2109x; 1.2109x over previous
import functools

import jax
import jax.numpy as jnp
from jax import lax
from jax.experimental import pallas as pl
from jax.experimental.pallas import tpu as pltpu
from jax.experimental.pallas import tpu_sc as plsc

NN = 100000
EE = 3200000
HID = 32
GG = 64

NC, NS = 2, 16
NW = NC * NS
NPAD = 100352
STRIPE = NPAD // NS
ROWS2D = NPAD // 128
RPW = 784
KT = 56
NT = RPW // KT
EPW = RPW * 128
EPAD = NW * EPW

RBLK = 2048
NSTEPS = NPAD // RBLK

_f32 = jnp.float32


def _zero_acc(sid, ztile, acc):
    @pl.loop(0, STRIPE, step=16)
    def _(i):
        ztile[pl.ds(i, 16)] = jnp.zeros((16,), _f32)

    pltpu.sync_copy(ztile, acc.at[pl.ds(sid * STRIPE, STRIPE)])


def _writeback(cid, sid, acc, out_hbm):
    pltpu.sync_copy(acc.at[pl.ds(sid * STRIPE, STRIPE)],
                    out_hbm.at[cid, pl.ds(sid * STRIPE, STRIPE)])


def _fire_scatters(val, idx, acc, sem):
    @pl.loop(0, KT)
    def _(j):
        pltpu.async_copy(val.at[j], acc.at[idx.at[j]], sem, add=True)


def _drain_scatters(val, idx, acc, sem):
    @pl.loop(0, KT)
    def _(j):
        pltpu.make_async_copy(val.at[j], acc.at[idx.at[j]], sem).wait()


def _fire_gathers(tab_hbm, idx, val, sem):
    @pl.loop(0, KT)
    def _(j):
        pltpu.async_copy(tab_hbm.at[idx.at[j]], val.at[j], sem)


def _drain_gathers(tab_hbm, idx, val, sem):
    @pl.loop(0, KT)
    def _(j):
        pltpu.make_async_copy(tab_hbm.at[idx.at[j]], val.at[j], sem).wait()


def _sc_hist(dst_hbm, out_hbm, idxa, idxb, ones_t, ztile, acc, sema, semb):
    cid = lax.axis_index("c")
    sid = lax.axis_index("s")
    wid = cid * NS + sid

    @pl.loop(0, 128, step=16)
    def _(i):
        ones_t[pl.ds(i, 16)] = jnp.ones((16,), _f32)

    _zero_acc(sid, ztile, acc)
    plsc.subcore_barrier()

    @pl.loop(0, NT, step=2)
    def _(t):
        pltpu.sync_copy(dst_hbm.at[wid, pl.ds(t * KT, KT)], idxa)

        @pl.loop(0, KT)
        def _(j):
            pltpu.async_copy(ones_t, acc.at[idxa.at[j]], sema, add=True)

        pltpu.sync_copy(dst_hbm.at[wid, pl.ds((t + 1) * KT, KT)], idxb)

        @pl.loop(0, KT)
        def _(j):
            pltpu.async_copy(ones_t, acc.at[idxb.at[j]], semb, add=True)

        @pl.loop(0, KT)
        def _(j):
            pltpu.make_async_copy(ones_t, acc.at[idxa.at[j]], sema).wait()

        @pl.loop(0, KT)
        def _(j):
            pltpu.make_async_copy(ones_t, acc.at[idxb.at[j]], semb).wait()

    plsc.subcore_barrier()
    _writeback(cid, sid, acc, out_hbm)


def _sc_agg1(src_hbm, dst_hbm, y_hbm, out_hbm,
             sidxa, sidxb, didxa, didxb, vala, valb, ztile, acc, ysh,
             gsema, gsemb, ssema, ssemb):
    cid = lax.axis_index("c")
    sid = lax.axis_index("s")
    wid = cid * NS + sid

    _zero_acc(sid, ztile, acc)
    pltpu.sync_copy(y_hbm.at[pl.ds(sid * STRIPE, STRIPE)], ztile)
    pltpu.sync_copy(ztile, ysh.at[pl.ds(sid * STRIPE, STRIPE)])
    plsc.subcore_barrier()

    @pl.loop(0, NT, step=2)
    def _(t):
        pltpu.sync_copy(src_hbm.at[wid, pl.ds(t * KT, KT)], sidxa)
        pltpu.sync_copy(dst_hbm.at[wid, pl.ds(t * KT, KT)], didxa)
        _fire_gathers(ysh, sidxa, vala, gsema)
        pltpu.sync_copy(src_hbm.at[wid, pl.ds((t + 1) * KT, KT)], sidxb)
        pltpu.sync_copy(dst_hbm.at[wid, pl.ds((t + 1) * KT, KT)], didxb)
        _fire_gathers(ysh, sidxb, valb, gsemb)
        _drain_gathers(ysh, sidxa, vala, gsema)
        _fire_scatters(vala, didxa, acc, ssema)
        _drain_gathers(ysh, sidxb, valb, gsemb)
        _fire_scatters(valb, didxb, acc, ssemb)
        _drain_scatters(vala, didxa, acc, ssema)
        _drain_scatters(valb, didxb, acc, ssemb)

    plsc.subcore_barrier()
    _writeback(cid, sid, acc, out_hbm)


def _sc_agg2(src_hbm, dst_hbm, g_hbm, outu_hbm, outv_hbm,
             sidxa, sidxb, didxa, didxb, valga, valgb, valza, valzb,
             ztile, accu, accv, gsh, gsema, gsemb, ssema, ssemb):
    cid = lax.axis_index("c")
    sid = lax.axis_index("s")
    wid = cid * NS + sid

    _zero_acc(sid, ztile, accu)
    _zero_acc(sid, ztile, accv)
    pltpu.sync_copy(g_hbm.at[pl.ds(sid * STRIPE, STRIPE)], ztile)
    pltpu.sync_copy(ztile, gsh.at[pl.ds(sid * STRIPE, STRIPE)])
    plsc.subcore_barrier()

    def relu_pair(valg, valz):
        @pl.loop(0, KT)
        def _(j):
            @pl.loop(0, 128, step=16)
            def _(i):
                vg = valg.at[j][pl.ds(i, 16)]
                valz.at[j][pl.ds(i, 16)] = jnp.maximum(-vg, 0.0)
                valg.at[j][pl.ds(i, 16)] = jnp.maximum(vg, 0.0)

    @pl.loop(0, NT, step=2)
    def _(t):
        pltpu.sync_copy(src_hbm.at[wid, pl.ds(t * KT, KT)], sidxa)
        pltpu.sync_copy(dst_hbm.at[wid, pl.ds(t * KT, KT)], didxa)
        _fire_gathers(gsh, sidxa, valga, gsema)
        pltpu.sync_copy(src_hbm.at[wid, pl.ds((t + 1) * KT, KT)], sidxb)
        pltpu.sync_copy(dst_hbm.at[wid, pl.ds((t + 1) * KT, KT)], didxb)
        _fire_gathers(gsh, sidxb, valgb, gsemb)
        _drain_gathers(gsh, sidxa, valga, gsema)
        relu_pair(valga, valza)
        _fire_scatters(valga, didxa, accu, ssema)
        _fire_scatters(valza, didxa, accv, ssema)
        _drain_gathers(gsh, sidxb, valgb, gsemb)
        relu_pair(valgb, valzb)
        _fire_scatters(valgb, didxb, accu, ssemb)
        _fire_scatters(valzb, didxb, accv, ssemb)
        _drain_scatters(valga, didxa, accu, ssema)
        _drain_scatters(valza, didxa, accv, ssema)
        _drain_scatters(valgb, didxb, accu, ssemb)
        _drain_scatters(valzb, didxb, accv, ssemb)

    plsc.subcore_barrier()
    _writeback(cid, sid, accu, outu_hbm)
    _writeback(cid, sid, accv, outv_hbm)


@functools.lru_cache(maxsize=1)
def _sc_kernels():
    mesh = plsc.VectorSubcoreMesh(core_axis_name="c", subcore_axis_name="s",
                                  num_cores=NC, num_subcores=NS)
    one_out = jax.ShapeDtypeStruct((NC, NPAD), _f32)
    idx_t = pltpu.VMEM((KT, 128), jnp.int32)
    val_t = pltpu.VMEM((KT, 128), _f32)
    dma = pltpu.SemaphoreType.DMA
    hist = pl.kernel(
        _sc_hist, out_type=one_out, mesh=mesh,
        scratch_types=[
            idx_t, idx_t,
            pltpu.VMEM((128,), _f32),
            pltpu.VMEM((STRIPE,), _f32),
            pltpu.VMEM_SHARED((NPAD,), _f32),
            dma, dma,
        ])
    agg1 = pl.kernel(
        _sc_agg1, out_type=one_out, mesh=mesh,
        scratch_types=[
            idx_t, idx_t, idx_t, idx_t,
            val_t, val_t,
            pltpu.VMEM((STRIPE,), _f32),
            pltpu.VMEM_SHARED((NPAD,), _f32),
            pltpu.VMEM_SHARED((NPAD,), _f32),
            dma, dma, dma, dma,
        ])
    agg2 = pl.kernel(
        _sc_agg2, out_type=[one_out, one_out], mesh=mesh,
        scratch_types=[
            idx_t, idx_t, idx_t, idx_t,
            val_t, val_t, val_t, val_t,
            pltpu.VMEM((STRIPE,), _f32),
            pltpu.VMEM_SHARED((NPAD,), _f32),
            pltpu.VMEM_SHARED((NPAD,), _f32),
            pltpu.VMEM_SHARED((NPAD,), _f32),
            dma, dma, dma, dma,
        ])
    return hist, agg1, agg2


def _tc_prep_body(degp, x2d, dinv_o, invd_o, y_o):
    deg = degp[0] + degp[1] + 1.0
    dinv = lax.rsqrt(deg)
    invd = 1.0 / deg
    dinv_o[...] = dinv
    invd_o[...] = invd
    y_o[...] = dinv * x2d[...]


def _tc_uvwz_body(tp, dinv, invd, x2d, u_o, v_o, g_o):
    t = dinv[...] * (tp[0] + tp[1]) + x2d[...] * invd[...]
    u_o[...] = jnp.maximum(t, 0.0)
    v_o[...] = jnp.maximum(-t, 0.0)
    g_o[...] = dinv[...] * t


def _tc_uv_body(up, vp, dinv, invd, u, v, uu_o, vv_o):
    uu_o[...] = dinv[...] * (up[0] + up[1]) + u[...] * invd[...]
    vv_o[...] = dinv[...] * (vp[0] + vp[1]) + v[...] * invd[...]


def _tc_pool_body(uu, vv, bat, a, c, b2, wl1, bl1, wl2, bl2, out,
                  sums, cnt):
    i = pl.program_id(0)

    @pl.when(i == 0)
    def _():
        sums[...] = jnp.zeros_like(sums)
        cnt[...] = jnp.zeros_like(cnt)

    h2 = jnp.maximum(uu[...] * a[...] + vv[...] * c[...] + b2[...], 0.0)
    seg = lax.broadcasted_iota(jnp.int32, (RBLK, GG), 1)
    onehot = (bat[...] == seg).astype(_f32)
    sums[...] += lax.dot_general(onehot, h2, (((0,), (0,)), ((), ())),
                                 preferred_element_type=_f32)
    cnt[...] += lax.dot_general(onehot, jnp.ones((RBLK, 1), _f32),
                                (((0,), (0,)), ((), ())),
                                preferred_element_type=_f32)

    @pl.when(i == NSTEPS - 1)
    def _():
        pooled = sums[...] / jnp.maximum(cnt[...], 1.0)
        hh = jnp.maximum(
            jnp.dot(pooled, wl1[...], preferred_element_type=_f32)
            + bl1[...], 0.0)
        oo = jnp.dot(hh, wl2[...], preferred_element_type=_f32) + bl2[...]
        out[...] = jax.nn.sigmoid(oo)


def kernel(x, edge_index, batch, W1, b1, W2, b2, Wl1, bl1, Wl2, bl2):
    f32 = _f32
    src = edge_index[0].astype(jnp.int32)
    dst = edge_index[1].astype(jnp.int32)
    npad_e = EPAD - EE
    src3 = jnp.concatenate([src, jnp.zeros((npad_e,), jnp.int32)]) \
        .reshape(NW, RPW, 128)
    dst3 = jnp.concatenate([dst, jnp.full((npad_e,), NN, jnp.int32)]) \
        .reshape(NW, RPW, 128)

    xp = jnp.concatenate([x[:, 0].astype(f32),
                          jnp.zeros((NPAD - NN,), f32)])
    x2d = xp.reshape(ROWS2D, 128)

    w1row = W1[0].astype(f32)
    p = jnp.maximum(w1row, 0.0)
    q = jnp.maximum(-w1row, 0.0)
    a = (p @ W2.astype(f32)).reshape(1, HID)
    c = (q @ W2.astype(f32)).reshape(1, HID)
    b2r = b2.astype(f32).reshape(1, HID)

    sc_hist, sc_agg1, sc_agg2 = _sc_kernels()

    degp = sc_hist(dst3)

    dinv2d, invd2d, y2d = pl.pallas_call(
        _tc_prep_body,
        out_shape=[jax.ShapeDtypeStruct((ROWS2D, 128), f32)] * 3,
    )(degp.reshape(NC, ROWS2D, 128), x2d)

    tp = sc_agg1(src3, dst3, y2d.reshape(NPAD))

    u2d, v2d, g2d = pl.pallas_call(
        _tc_uvwz_body,
        out_shape=[jax.ShapeDtypeStruct((ROWS2D, 128), f32)] * 3,
    )(tp.reshape(NC, ROWS2D, 128), dinv2d, invd2d, x2d)

    up, vp = sc_agg2(src3, dst3, g2d.reshape(NPAD))

    uu2d, vv2d = pl.pallas_call(
        _tc_uv_body,
        out_shape=[jax.ShapeDtypeStruct((ROWS2D, 128), f32)] * 2,
    )(up.reshape(NC, ROWS2D, 128), vp.reshape(NC, ROWS2D, 128),
      dinv2d, invd2d, u2d, v2d)

    batp = jnp.concatenate([batch.astype(jnp.int32),
                            jnp.full((NPAD - NN,), GG, jnp.int32)]) \
        .reshape(NPAD, 1)

    out = pl.pallas_call(
        _tc_pool_body,
        grid=(NSTEPS,),
        in_specs=[
            pl.BlockSpec((RBLK, 1), lambda i: (i, 0)),
            pl.BlockSpec((RBLK, 1), lambda i: (i, 0)),
            pl.BlockSpec((RBLK, 1), lambda i: (i, 0)),
            pl.BlockSpec((1, HID), lambda i: (0, 0)),
            pl.BlockSpec((1, HID), lambda i: (0, 0)),
            pl.BlockSpec((1, HID), lambda i: (0, 0)),
            pl.BlockSpec((HID, 16), lambda i: (0, 0)),
            pl.BlockSpec((1, 16), lambda i: (0, 0)),
            pl.BlockSpec((16, 1), lambda i: (0, 0)),
            pl.BlockSpec((1, 1), lambda i: (0, 0)),
        ],
        out_specs=pl.BlockSpec((GG, 1), lambda i: (0, 0)),
        out_shape=jax.ShapeDtypeStruct((GG, 1), f32),
        scratch_shapes=[pltpu.VMEM((GG, HID), f32),
                        pltpu.VMEM((GG, 1), f32)],
    )(uu2d.reshape(NPAD, 1), vv2d.reshape(NPAD, 1), batp,
      a, c, b2r,
      Wl1.astype(f32), bl1.astype(f32).reshape(1, 16),
      Wl2.astype(f32), bl2.astype(f32).reshape(1, 1))

    return out.reshape(GG)


# speedup vs baseline: 1.3106x; 1.0823x over previous
import functools

import jax
import jax.numpy as jnp
from jax import lax
from jax.experimental import pallas as pl
from jax.experimental.pallas import tpu as pltpu
from jax.experimental.pallas import tpu_sc as plsc

NN = 100000
EE = 3200000
HID = 32
GG = 64

NC, NS = 2, 16
NW = NC * NS
NPAD = 100352
STRIPE = NPAD // NS
ROWS2D = NPAD // 128
RPW = 784
KT = 56
NT = RPW // KT
EPW = RPW * 128
EPAD = NW * EPW

RBLK = 2048
NSTEPS = NPAD // RBLK

_f32 = jnp.float32


def _zero_acc(sid, ztile, acc):
    @pl.loop(0, STRIPE, step=16)
    def _(i):
        ztile[pl.ds(i, 16)] = jnp.zeros((16,), _f32)

    pltpu.sync_copy(ztile, acc.at[pl.ds(sid * STRIPE, STRIPE)])


def _writeback(cid, sid, acc, out_hbm):
    pltpu.sync_copy(acc.at[pl.ds(sid * STRIPE, STRIPE)],
                    out_hbm.at[cid, pl.ds(sid * STRIPE, STRIPE)])


def _fire_scatters(val, idx, acc, sem):
    @pl.loop(0, KT)
    def _(j):
        pltpu.async_copy(val.at[j], acc.at[idx.at[j]], sem, add=True)


def _drain_scatters(val, idx, acc, sem):
    @pl.loop(0, KT)
    def _(j):
        pltpu.make_async_copy(val.at[j], acc.at[idx.at[j]], sem).wait()


def _fire_gathers(tab_hbm, idx, val, sem):
    @pl.loop(0, KT)
    def _(j):
        pltpu.async_copy(tab_hbm.at[idx.at[j]], val.at[j], sem)


def _drain_gathers(tab_hbm, idx, val, sem):
    @pl.loop(0, KT)
    def _(j):
        pltpu.make_async_copy(tab_hbm.at[idx.at[j]], val.at[j], sem).wait()


def _sc_hist(dst_hbm, out_hbm, idxa, idxb, ones_t, ztile, acc, sema, semb):
    cid = lax.axis_index("c")
    sid = lax.axis_index("s")
    wid = cid * NS + sid

    @pl.loop(0, 128, step=16)
    def _(i):
        ones_t[pl.ds(i, 16)] = jnp.ones((16,), _f32)

    _zero_acc(sid, ztile, acc)
    plsc.subcore_barrier()

    @pl.loop(0, NT, step=2)
    def _(t):
        pltpu.sync_copy(dst_hbm.at[wid, pl.ds(t * KT, KT)], idxa)

        @pl.loop(0, KT)
        def _(j):
            pltpu.async_copy(ones_t, acc.at[idxa.at[j]], sema, add=True)

        pltpu.sync_copy(dst_hbm.at[wid, pl.ds((t + 1) * KT, KT)], idxb)

        @pl.loop(0, KT)
        def _(j):
            pltpu.async_copy(ones_t, acc.at[idxb.at[j]], semb, add=True)

        @pl.loop(0, KT)
        def _(j):
            pltpu.make_async_copy(ones_t, acc.at[idxa.at[j]], sema).wait()

        @pl.loop(0, KT)
        def _(j):
            pltpu.make_async_copy(ones_t, acc.at[idxb.at[j]], semb).wait()

    plsc.subcore_barrier()
    _writeback(cid, sid, acc, out_hbm)


def _sc_agg1(src_hbm, dst_hbm, y_hbm, out_hbm,
             sidxa, sidxb, didxa, didxb, vala, valb, ztile, acc, ysh,
             gsema, gsemb, ssema, ssemb):
    cid = lax.axis_index("c")
    sid = lax.axis_index("s")
    wid = cid * NS + sid

    _zero_acc(sid, ztile, acc)
    pltpu.sync_copy(y_hbm.at[pl.ds(sid * STRIPE, STRIPE)], ztile)
    pltpu.sync_copy(ztile, ysh.at[pl.ds(sid * STRIPE, STRIPE)])
    plsc.subcore_barrier()

    @pl.loop(0, NT, step=2)
    def _(t):
        pltpu.sync_copy(src_hbm.at[wid, pl.ds(t * KT, KT)], sidxa)
        pltpu.sync_copy(dst_hbm.at[wid, pl.ds(t * KT, KT)], didxa)
        _fire_gathers(ysh, sidxa, vala, gsema)
        pltpu.sync_copy(src_hbm.at[wid, pl.ds((t + 1) * KT, KT)], sidxb)
        pltpu.sync_copy(dst_hbm.at[wid, pl.ds((t + 1) * KT, KT)], didxb)
        _fire_gathers(ysh, sidxb, valb, gsemb)
        _drain_gathers(ysh, sidxa, vala, gsema)
        _fire_scatters(vala, didxa, acc, ssema)
        _drain_gathers(ysh, sidxb, valb, gsemb)
        _fire_scatters(valb, didxb, acc, ssemb)
        _drain_scatters(vala, didxa, acc, ssema)
        _drain_scatters(valb, didxb, acc, ssemb)

    plsc.subcore_barrier()
    _writeback(cid, sid, acc, out_hbm)


def _sc_agg2(src_hbm, dst_hbm, g_hbm, out_hbm,
             sidxa, sidxb, didxa, didxb, idx2a, idx2b, valga, valgb,
             ztile, accc, gsh, gsema, gsemb, ssema, ssemb):
    cid = lax.axis_index("c")
    sid = lax.axis_index("s")
    wid = cid * NS + sid

    @pl.loop(0, STRIPE, step=16)
    def _(i):
        ztile[pl.ds(i, 16)] = jnp.zeros((16,), _f32)

    pltpu.sync_copy(ztile, accc.at[pl.ds(sid * STRIPE, STRIPE)])
    pltpu.sync_copy(ztile, accc.at[pl.ds(NPAD + sid * STRIPE, STRIPE)])
    pltpu.sync_copy(g_hbm.at[pl.ds(sid * STRIPE, STRIPE)], ztile)
    pltpu.sync_copy(ztile, gsh.at[pl.ds(sid * STRIPE, STRIPE)])
    plsc.subcore_barrier()

    def transform(valg, didx, idx2):
        @pl.loop(0, KT)
        def _(j):
            @pl.loop(0, 128, step=16)
            def _(i):
                vg = valg.at[j][pl.ds(i, 16)]
                off = jnp.where(vg < 0.0, NPAD, 0).astype(jnp.int32)
                idx2.at[j][pl.ds(i, 16)] = \
                    didx.at[j][pl.ds(i, 16)] + off
                valg.at[j][pl.ds(i, 16)] = jnp.abs(vg)

    @pl.loop(0, NT, step=2)
    def _(t):
        pltpu.sync_copy(src_hbm.at[wid, pl.ds(t * KT, KT)], sidxa)
        pltpu.sync_copy(dst_hbm.at[wid, pl.ds(t * KT, KT)], didxa)
        _fire_gathers(gsh, sidxa, valga, gsema)
        pltpu.sync_copy(src_hbm.at[wid, pl.ds((t + 1) * KT, KT)], sidxb)
        pltpu.sync_copy(dst_hbm.at[wid, pl.ds((t + 1) * KT, KT)], didxb)
        _fire_gathers(gsh, sidxb, valgb, gsemb)
        _drain_gathers(gsh, sidxa, valga, gsema)
        transform(valga, didxa, idx2a)
        _fire_scatters(valga, idx2a, accc, ssema)
        _drain_gathers(gsh, sidxb, valgb, gsemb)
        transform(valgb, didxb, idx2b)
        _fire_scatters(valgb, idx2b, accc, ssemb)
        _drain_scatters(valga, idx2a, accc, ssema)
        _drain_scatters(valgb, idx2b, accc, ssemb)

    plsc.subcore_barrier()
    pltpu.sync_copy(accc.at[pl.ds(sid * STRIPE, STRIPE)],
                    out_hbm.at[cid, 0, pl.ds(sid * STRIPE, STRIPE)])
    pltpu.sync_copy(accc.at[pl.ds(NPAD + sid * STRIPE, STRIPE)],
                    out_hbm.at[cid, 1, pl.ds(sid * STRIPE, STRIPE)])


@functools.lru_cache(maxsize=1)
def _sc_kernels():
    mesh = plsc.VectorSubcoreMesh(core_axis_name="c", subcore_axis_name="s",
                                  num_cores=NC, num_subcores=NS)
    one_out = jax.ShapeDtypeStruct((NC, NPAD), _f32)
    idx_t = pltpu.VMEM((KT, 128), jnp.int32)
    val_t = pltpu.VMEM((KT, 128), _f32)
    dma = pltpu.SemaphoreType.DMA
    hist = pl.kernel(
        _sc_hist, out_type=one_out, mesh=mesh,
        scratch_types=[
            idx_t, idx_t,
            pltpu.VMEM((128,), _f32),
            pltpu.VMEM((STRIPE,), _f32),
            pltpu.VMEM_SHARED((NPAD,), _f32),
            dma, dma,
        ])
    agg1 = pl.kernel(
        _sc_agg1, out_type=one_out, mesh=mesh,
        scratch_types=[
            idx_t, idx_t, idx_t, idx_t,
            val_t, val_t,
            pltpu.VMEM((STRIPE,), _f32),
            pltpu.VMEM_SHARED((NPAD,), _f32),
            pltpu.VMEM_SHARED((NPAD,), _f32),
            dma, dma, dma, dma,
        ])
    agg2 = pl.kernel(
        _sc_agg2, out_type=jax.ShapeDtypeStruct((NC, 2, NPAD), _f32),
        mesh=mesh,
        scratch_types=[
            idx_t, idx_t, idx_t, idx_t, idx_t, idx_t,
            val_t, val_t,
            pltpu.VMEM((STRIPE,), _f32),
            pltpu.VMEM_SHARED((2 * NPAD,), _f32),
            pltpu.VMEM_SHARED((NPAD,), _f32),
            dma, dma, dma, dma,
        ])
    return hist, agg1, agg2


def _tc_prep_body(degp, x2d, dinv_o, invd_o, y_o):
    deg = degp[0] + degp[1] + 1.0
    dinv = lax.rsqrt(deg)
    invd = 1.0 / deg
    dinv_o[...] = dinv
    invd_o[...] = invd
    y_o[...] = dinv * x2d[...]


def _tc_uvwz_body(tp, dinv, invd, x2d, u_o, v_o, g_o):
    t = dinv[...] * (tp[0] + tp[1]) + x2d[...] * invd[...]
    u_o[...] = jnp.maximum(t, 0.0)
    v_o[...] = jnp.maximum(-t, 0.0)
    g_o[...] = dinv[...] * t


def _tc_uv_body(up, vp, dinv, invd, u, v, uu_o, vv_o):
    uu_o[...] = dinv[...] * (up[0] + up[1]) + u[...] * invd[...]
    vv_o[...] = dinv[...] * (vp[0] + vp[1]) + v[...] * invd[...]


def _tc_pool_body(uu, vv, bat, a, c, b2, wl1, bl1, wl2, bl2, out,
                  sums, cnt):
    i = pl.program_id(0)

    @pl.when(i == 0)
    def _():
        sums[...] = jnp.zeros_like(sums)
        cnt[...] = jnp.zeros_like(cnt)

    h2 = jnp.maximum(uu[...] * a[...] + vv[...] * c[...] + b2[...], 0.0)
    seg = lax.broadcasted_iota(jnp.int32, (RBLK, GG), 1)
    onehot = (bat[...] == seg).astype(_f32)
    sums[...] += lax.dot_general(onehot, h2, (((0,), (0,)), ((), ())),
                                 preferred_element_type=_f32)
    cnt[...] += lax.dot_general(onehot, jnp.ones((RBLK, 1), _f32),
                                (((0,), (0,)), ((), ())),
                                preferred_element_type=_f32)

    @pl.when(i == NSTEPS - 1)
    def _():
        pooled = sums[...] / jnp.maximum(cnt[...], 1.0)
        hh = jnp.maximum(
            jnp.dot(pooled, wl1[...], preferred_element_type=_f32)
            + bl1[...], 0.0)
        oo = jnp.dot(hh, wl2[...], preferred_element_type=_f32) + bl2[...]
        out[...] = jax.nn.sigmoid(oo)


def kernel(x, edge_index, batch, W1, b1, W2, b2, Wl1, bl1, Wl2, bl2):
    f32 = _f32
    src = edge_index[0].astype(jnp.int32)
    dst = edge_index[1].astype(jnp.int32)
    npad_e = EPAD - EE
    src3 = jnp.concatenate([src, jnp.zeros((npad_e,), jnp.int32)]) \
        .reshape(NW, RPW, 128)
    dst3 = jnp.concatenate([dst, jnp.full((npad_e,), NN, jnp.int32)]) \
        .reshape(NW, RPW, 128)

    xp = jnp.concatenate([x[:, 0].astype(f32),
                          jnp.zeros((NPAD - NN,), f32)])
    x2d = xp.reshape(ROWS2D, 128)

    w1row = W1[0].astype(f32)
    p = jnp.maximum(w1row, 0.0)
    q = jnp.maximum(-w1row, 0.0)
    a = (p @ W2.astype(f32)).reshape(1, HID)
    c = (q @ W2.astype(f32)).reshape(1, HID)
    b2r = b2.astype(f32).reshape(1, HID)

    sc_hist, sc_agg1, sc_agg2 = _sc_kernels()

    degp = sc_hist(dst3)

    dinv2d, invd2d, y2d = pl.pallas_call(
        _tc_prep_body,
        out_shape=[jax.ShapeDtypeStruct((ROWS2D, 128), f32)] * 3,
    )(degp.reshape(NC, ROWS2D, 128), x2d)

    tp = sc_agg1(src3, dst3, y2d.reshape(NPAD))

    u2d, v2d, g2d = pl.pallas_call(
        _tc_uvwz_body,
        out_shape=[jax.ShapeDtypeStruct((ROWS2D, 128), f32)] * 3,
    )(tp.reshape(NC, ROWS2D, 128), dinv2d, invd2d, x2d)

    upvp = sc_agg2(src3, dst3, g2d.reshape(NPAD))
    up = upvp[:, 0]
    vp = upvp[:, 1]

    uu2d, vv2d = pl.pallas_call(
        _tc_uv_body,
        out_shape=[jax.ShapeDtypeStruct((ROWS2D, 128), f32)] * 2,
    )(up.reshape(NC, ROWS2D, 128), vp.reshape(NC, ROWS2D, 128),
      dinv2d, invd2d, u2d, v2d)

    batp = jnp.concatenate([batch.astype(jnp.int32),
                            jnp.full((NPAD - NN,), GG, jnp.int32)]) \
        .reshape(NPAD, 1)

    out = pl.pallas_call(
        _tc_pool_body,
        grid=(NSTEPS,),
        in_specs=[
            pl.BlockSpec((RBLK, 1), lambda i: (i, 0)),
            pl.BlockSpec((RBLK, 1), lambda i: (i, 0)),
            pl.BlockSpec((RBLK, 1), lambda i: (i, 0)),
            pl.BlockSpec((1, HID), lambda i: (0, 0)),
            pl.BlockSpec((1, HID), lambda i: (0, 0)),
            pl.BlockSpec((1, HID), lambda i: (0, 0)),
            pl.BlockSpec((HID, 16), lambda i: (0, 0)),
            pl.BlockSpec((1, 16), lambda i: (0, 0)),
            pl.BlockSpec((16, 1), lambda i: (0, 0)),
            pl.BlockSpec((1, 1), lambda i: (0, 0)),
        ],
        out_specs=pl.BlockSpec((GG, 1), lambda i: (0, 0)),
        out_shape=jax.ShapeDtypeStruct((GG, 1), f32),
        scratch_shapes=[pltpu.VMEM((GG, HID), f32),
                        pltpu.VMEM((GG, 1), f32)],
    )(uu2d.reshape(NPAD, 1), vv2d.reshape(NPAD, 1), batp,
      a, c, b2r,
      Wl1.astype(f32), bl1.astype(f32).reshape(1, 16),
      Wl2.astype(f32), bl2.astype(f32).reshape(1, 1))

    return out.reshape(GG)
